# baseline (speedup 1.0000x reference)
import functools
import math

import jax
import jax.numpy as jnp
from jax import lax
from jax.experimental import pallas as pl
from jax.experimental.pallas import tpu as pltpu
from jax.experimental.pallas import tpu_sc as plsc

N_NODES = 10000
N_EDGES = 320000
DIM = 128
EDGE_DIM = 16
HEADS = 4
DH = DIM // HEADS

QG_W = DIM + HEADS * EDGE_DIM
KV_W = 2 * DIM
ACC_W = 144

NC = 2
NS = 16
NW = NC * NS
EPW = N_EDGES // NW
CHUNK = 16
N_CHUNKS = EPW // CHUNK

N_PAD = 10240
ROWS_PER_TILE = N_PAD // NS
ZROWS = 80


NBUF = 3


def _sc_conv(qg, kv, ea, seg, oth):
    mesh = plsc.VectorSubcoreMesh(core_axis_name="c", subcore_axis_name="s")

    scratch = (
        [pltpu.VMEM((CHUNK,), jnp.int32) for _ in range(3 * NBUF)]
        + [pltpu.VMEM((CHUNK, EDGE_DIM), jnp.float32) for _ in range(NBUF)]
        + [pltpu.VMEM((CHUNK, QG_W), jnp.float32) for _ in range(NBUF)]
        + [pltpu.VMEM((CHUNK, KV_W), jnp.float32) for _ in range(NBUF)]
        + [pltpu.VMEM((CHUNK, ACC_W), jnp.float32) for _ in range(NBUF)]
        + [pltpu.VMEM_SHARED((N_PAD, ACC_W), jnp.float32)]
        + [pltpu.SemaphoreType.DMA for _ in range(3 * NBUF)]
    )

    @functools.partial(
        pl.kernel,
        out_type=jax.ShapeDtypeStruct((NC * N_PAD, ACC_W), jnp.float32),
        mesh=mesh,
        compiler_params=pltpu.CompilerParams(
            needs_layout_passes=False, use_tc_tiling_on_sc=False),
        scratch_types=scratch,
    )
    def conv(qg_hbm, kv_hbm, ea_hbm, seg_hbm, oth_hbm, out_hbm, *bufs):
        segs = bufs[0:NBUF]
        oths = bufs[NBUF:2 * NBUF]
        sidx = bufs[2 * NBUF:3 * NBUF]
        eabs = bufs[3 * NBUF:4 * NBUF]
        qgs = bufs[4 * NBUF:5 * NBUF]
        kvs = bufs[5 * NBUF:6 * NBUF]
        wvs = bufs[6 * NBUF:7 * NBUF]
        acc_sh = bufs[7 * NBUF]
        semi = bufs[7 * NBUF + 1:7 * NBUF + 1 + NBUF]
        semg = bufs[7 * NBUF + 1 + NBUF:7 * NBUF + 1 + 2 * NBUF]
        sems = bufs[7 * NBUF + 1 + 2 * NBUF:7 * NBUF + 1 + 3 * NBUF]

        cid = lax.axis_index("c")
        sid = lax.axis_index("s")
        wid = sid * NC + cid
        ebase = wid * EPW

        zero16 = jnp.zeros((16,), jnp.float32)

        for b in range(NBUF):
            for r in range(CHUNK):
                for c9 in range(ACC_W // 16):
                    wvs[b][r, pl.ds(c9 * 16, 16)] = zero16

        for t in range(ROWS_PER_TILE // CHUNK):
            pltpu.sync_copy(
                wvs[0], acc_sh.at[pl.ds(sid * ROWS_PER_TILE + t * CHUNK, CHUNK)])
        plsc.subcore_barrier()

        lane = lax.iota(jnp.int32, 16)
        inv_sqrt = jnp.float32(1.0 / math.sqrt(DH))

        def idx_issue(ci, b):
            base = ebase + ci * CHUNK
            pltpu.async_copy(seg_hbm.at[pl.ds(base, CHUNK)], segs[b], semi[b])
            pltpu.async_copy(oth_hbm.at[pl.ds(base, CHUNK)], oths[b], semi[b])
            pltpu.async_copy(ea_hbm.at[pl.ds(base, CHUNK)], eabs[b], semi[b])

        def idx_wait(b):
            pltpu.make_async_copy(
                seg_hbm.at[pl.ds(0, CHUNK)], segs[b], semi[b]).wait()
            pltpu.make_async_copy(
                oth_hbm.at[pl.ds(0, CHUNK)], oths[b], semi[b]).wait()
            pltpu.make_async_copy(
                ea_hbm.at[pl.ds(0, CHUNK)], eabs[b], semi[b]).wait()

        def gather_issue(b):
            pltpu.async_copy(qg_hbm.at[segs[b]], qgs[b], semg[b])
            pltpu.async_copy(kv_hbm.at[oths[b]], kvs[b], semg[b])

        def gather_wait(b):
            pltpu.make_async_copy(qg_hbm.at[segs[b]], qgs[b], semg[b]).wait()
            pltpu.make_async_copy(kv_hbm.at[oths[b]], kvs[b], semg[b]).wait()

        def compute(b):
            qgv, kvv, eav, wvv = qgs[b], kvs[b], eabs[b], wvs[b]

            def qk_step(s2, accs):
                accs = list(accs)
                for u in range(4):
                    s = s2 * 4 + u
                    rot = jnp.bitwise_and(lane + s, DH - 1)
                    for h in range(HEADS):
                        col = rot + (h * DH)
                        qv = plsc.load_gather(qgv, [lane, col])
                        kx = plsc.load_gather(kvv, [lane, col])
                        accs[h] = accs[h] + qv * kx
                return tuple(accs)

            z = jnp.zeros((16,), jnp.float32)
            accs = lax.fori_loop(0, DH // 4, qk_step, (z, z, z, z))

            def g_step(s2, accs):
                accs = list(accs)
                for u in range(4):
                    s = s2 * 4 + u
                    c16 = jnp.bitwise_and(lane + s, EDGE_DIM - 1)
                    eav_s = plsc.load_gather(eav, [lane, c16])
                    for h in range(HEADS):
                        gv = plsc.load_gather(
                            qgv, [lane, c16 + (DIM + h * EDGE_DIM)])
                        accs[h] = accs[h] + gv * eav_s
                return tuple(accs)

            accs = lax.fori_loop(0, EDGE_DIM // 4, g_step, accs)
            ws = tuple(jnp.exp(a * inv_sqrt) for a in accs)
            for h in range(HEADS):
                plsc.store_scatter(
                    wvv, [lane, jnp.full((16,), DIM + h, jnp.int32)], ws[h])

            def wv_step(s2, wsc):
                for u in range(4):
                    s = s2 * 4 + u
                    rot = jnp.bitwise_and(lane + s, DH - 1)
                    for h in range(HEADS):
                        col = rot + (h * DH)
                        vv = plsc.load_gather(kvv, [lane, col + DIM])
                        plsc.store_scatter(wvv, [lane, col], vv * wsc[h])
                return wsc

            lax.fori_loop(0, DH // 4, wv_step, ws)

        def scatter_issue(b):
            sidx[b][...] = segs[b][...]
            pltpu.async_copy(wvs[b], acc_sh.at[sidx[b]], sems[b], add=True)

        def scatter_wait(b):
            pltpu.make_async_copy(wvs[b], acc_sh.at[sidx[b]], sems[b]).wait()

        for b in range(NBUF):
            idx_issue(b, b)
        idx_wait(0)
        gather_issue(0)
        idx_wait(1)
        gather_issue(1)

        def pipe_body(o, carry):
            for b in range(NBUF):
                ci = o * NBUF + b
                nb = (b + 2) % NBUF
                gather_wait(b)

                @pl.when(ci + 2 <= N_CHUNKS - 1)
                def _():
                    idx_wait(nb)
                    gather_issue(nb)

                @pl.when(ci >= NBUF)
                def _():
                    scatter_wait(b)
                compute(b)
                scatter_issue(b)

                @pl.when(ci + NBUF <= N_CHUNKS - 1)
                def _():
                    idx_issue(ci + NBUF, b)
            return carry
        lax.fori_loop(0, (N_CHUNKS - 1) // NBUF, pipe_body, 0)

        gather_wait(0)
        scatter_wait(0)
        compute(0)
        scatter_issue(0)
        scatter_wait(1)
        scatter_wait(2)
        scatter_wait(0)

        plsc.subcore_barrier()
        for t in range(ROWS_PER_TILE // 160):
            r0 = sid * ROWS_PER_TILE + t * 160
            pltpu.sync_copy(
                acc_sh.at[pl.ds(r0, 160)],
                out_hbm.at[pl.ds(cid * N_PAD + r0, 160)])

    return conv(qg, kv, ea, seg, oth)


ROWS_BLK = 1000
N_BLK = N_NODES // ROWS_BLK
PREP_W = 2 * (QG_W + KV_W)


def _prep(h, W, b):
    def body(h_ref, w_ref, b_ref, qgf_ref, kvf_ref, qgb_ref, kvb_ref):
        res = jnp.dot(h_ref[...], w_ref[...],
                      preferred_element_type=jnp.float32) + b_ref[...]
        o = 0
        qgf_ref[...] = res[:, o:o + QG_W]; o += QG_W
        kvf_ref[...] = res[:, o:o + KV_W]; o += KV_W
        qgb_ref[...] = res[:, o:o + QG_W]; o += QG_W
        kvb_ref[...] = res[:, o:o + KV_W]

    qg_spec = pl.BlockSpec((ROWS_BLK, QG_W), lambda i: (i, 0))
    kv_spec = pl.BlockSpec((ROWS_BLK, KV_W), lambda i: (i, 0))
    qg_shape = jax.ShapeDtypeStruct((N_NODES, QG_W), jnp.float32)
    kv_shape = jax.ShapeDtypeStruct((N_NODES, KV_W), jnp.float32)
    return pl.pallas_call(
        body,
        grid=(N_BLK,),
        in_specs=[
            pl.BlockSpec((ROWS_BLK, DIM), lambda i: (i, 0)),
            pl.BlockSpec((DIM, PREP_W), lambda i: (0, 0)),
            pl.BlockSpec((PREP_W,), lambda i: (0,)),
        ],
        out_specs=[qg_spec, kv_spec, qg_spec, kv_spec],
        out_shape=[qg_shape, kv_shape, qg_shape, kv_shape],
    )(h, W, b)


def _merge(h, accf, accb, mW, ln_g, ln_b, sel, e144):
    def body(h_ref, af0, af1, ab0, ab1, mw_ref, g_ref, b_ref, sel_ref,
             e_ref, o_ref):
        af = af0[0] + af1[0]
        ab = ab0[0] + ab1[0]
        s = sel_ref[...]
        e = e_ref[...]
        fw = jnp.dot(af, s, preferred_element_type=jnp.float32)
        fden = jnp.dot(af, e, preferred_element_type=jnp.float32)
        bw = jnp.dot(ab, s, preferred_element_type=jnp.float32)
        bden = jnp.dot(ab, e, preferred_element_type=jnp.float32)
        fw = fw / (fden + 1e-16)
        bw = bw / (bden + 1e-16)
        mw = mw_ref[...]
        merged = (jnp.dot(h_ref[...], mw[:DIM],
                          preferred_element_type=jnp.float32)
                  + jnp.dot(fw, mw[DIM:2 * DIM],
                            preferred_element_type=jnp.float32)
                  + jnp.dot(bw, mw[2 * DIM:],
                            preferred_element_type=jnp.float32))
        gel = 0.5 * merged * (1.0 + lax.erf(merged * jnp.float32(1.0 / math.sqrt(2.0))))
        mu = jnp.mean(gel, axis=-1, keepdims=True)
        var = jnp.mean((gel - mu) ** 2, axis=-1, keepdims=True)
        o_ref[...] = (gel - mu) / jnp.sqrt(var + 1e-5) * g_ref[...] + b_ref[...]

    acc_spec0 = pl.BlockSpec((1, ROWS_BLK, ACC_W), lambda i: (0, i, 0))
    acc_spec1 = pl.BlockSpec((1, ROWS_BLK, ACC_W), lambda i: (1, i, 0))
    return pl.pallas_call(
        body,
        grid=(N_BLK,),
        in_specs=[
            pl.BlockSpec((ROWS_BLK, DIM), lambda i: (i, 0)),
            acc_spec0, acc_spec1, acc_spec0, acc_spec1,
            pl.BlockSpec((3 * DIM, DIM), lambda i: (0, 0)),
            pl.BlockSpec((DIM,), lambda i: (0,)),
            pl.BlockSpec((DIM,), lambda i: (0,)),
            pl.BlockSpec((ACC_W, DIM), lambda i: (0, 0)),
            pl.BlockSpec((ACC_W, DIM), lambda i: (0, 0)),
        ],
        out_specs=pl.BlockSpec((ROWS_BLK, DIM), lambda i: (i, 0)),
        out_shape=jax.ShapeDtypeStruct((N_NODES, DIM), jnp.float32),
    )(h, accf, accf, accb, accb, mW, ln_g, ln_b, sel, e144)


def _build_kwe(kW):
    kwe_part = kW[DIM:]
    out = jnp.zeros((DIM, HEADS * EDGE_DIM), jnp.float32)
    for hh in range(HEADS):
        blk = kwe_part[:, hh * DH:(hh + 1) * DH].T
        out = out.at[hh * DH:(hh + 1) * DH,
                     hh * EDGE_DIM:(hh + 1) * EDGE_DIM].set(blk)
    return out


def _pack_dir(p):
    kwe = _build_kwe(p["kW"])
    wqg = jnp.concatenate([p["qW"], p["qW"] @ kwe], axis=1)
    bqg = jnp.concatenate([p["qb"], p["qb"] @ kwe], axis=0)
    wkv = jnp.concatenate([p["kW"][:DIM], p["vW"]], axis=1)
    bkv = jnp.concatenate([p["kb"], p["vb"]], axis=0)
    return jnp.concatenate([wqg, wkv], axis=1), jnp.concatenate([bqg, bkv])


def kernel(x, edge_index, edge_attr, params):
    src = edge_index[0]
    dst = edge_index[1]

    sel = jnp.concatenate(
        [jnp.eye(DIM, dtype=jnp.float32),
         jnp.zeros((ACC_W - DIM, DIM), jnp.float32)], axis=0)
    rep = jnp.kron(jnp.eye(HEADS, dtype=jnp.float32),
                   jnp.ones((1, DH), jnp.float32))
    e144 = jnp.concatenate(
        [jnp.zeros((DIM, DIM), jnp.float32), rep,
         jnp.zeros((ACC_W - DIM - HEADS, DIM), jnp.float32)], axis=0)

    h = x
    for p in params["layers"]:
        wf, bf = _pack_dir(p["fw"])
        wb, bb = _pack_dir(p["bw"])
        W = jnp.concatenate([wf, wb], axis=1)
        b = jnp.concatenate([bf, bb])
        qgf, kvf, qgb, kvb = _prep(h, W, b)
        accf = _sc_conv(qgf, kvf, edge_attr, dst, src).reshape(NC, N_PAD, ACC_W)
        accb = _sc_conv(qgb, kvb, edge_attr, src, dst).reshape(NC, N_PAD, ACC_W)
        h = _merge(h, accf, accb, p["mW"], p["ln_g"], p["ln_b"], sel, e144)
    return h


# speedup vs baseline: 1.0230x; 1.0230x over previous
import functools
import math

import jax
import jax.numpy as jnp
from jax import lax
from jax.experimental import pallas as pl
from jax.experimental.pallas import tpu as pltpu
from jax.experimental.pallas import tpu_sc as plsc

N_NODES = 10000
N_EDGES = 320000
DIM = 128
EDGE_DIM = 16
HEADS = 4
DH = DIM // HEADS

QG_W = DIM + HEADS * EDGE_DIM
KV_W = 2 * DIM
ACC_W = 144

NC = 2
NS = 16
NW = NC * NS
EPW = N_EDGES // NW
CHUNK = 16
N_CHUNKS = EPW // CHUNK

N_PAD = 10240
ROWS_PER_TILE = N_PAD // NS
ZROWS = 80


NBUF = 2


def _sc_conv(qg, kv, ea, seg, oth):
    mesh = plsc.VectorSubcoreMesh(core_axis_name="c", subcore_axis_name="s")

    scratch = (
        [pltpu.VMEM((CHUNK,), jnp.int32) for _ in range(3 * NBUF)]
        + [pltpu.VMEM((CHUNK, EDGE_DIM), jnp.float32) for _ in range(NBUF)]
        + [pltpu.VMEM((CHUNK, QG_W), jnp.float32) for _ in range(NBUF)]
        + [pltpu.VMEM((CHUNK, KV_W), jnp.float32) for _ in range(NBUF)]
        + [pltpu.VMEM((CHUNK, ACC_W), jnp.float32) for _ in range(NBUF)]
        + [pltpu.VMEM_SHARED((N_PAD, ACC_W), jnp.float32)]
        + [pltpu.SemaphoreType.DMA for _ in range(3 * NBUF)]
    )

    @functools.partial(
        pl.kernel,
        out_type=jax.ShapeDtypeStruct((NC * N_PAD, ACC_W), jnp.float32),
        mesh=mesh,
        compiler_params=pltpu.CompilerParams(
            needs_layout_passes=False, use_tc_tiling_on_sc=False),
        scratch_types=scratch,
    )
    def conv(qg_hbm, kv_hbm, ea_hbm, seg_hbm, oth_hbm, out_hbm, *bufs):
        segs = bufs[0:NBUF]
        oths = bufs[NBUF:2 * NBUF]
        sidx = bufs[2 * NBUF:3 * NBUF]
        eabs = bufs[3 * NBUF:4 * NBUF]
        qgs = bufs[4 * NBUF:5 * NBUF]
        kvs = bufs[5 * NBUF:6 * NBUF]
        wvs = bufs[6 * NBUF:7 * NBUF]
        acc_sh = bufs[7 * NBUF]
        semi = bufs[7 * NBUF + 1:7 * NBUF + 1 + NBUF]
        semg = bufs[7 * NBUF + 1 + NBUF:7 * NBUF + 1 + 2 * NBUF]
        sems = bufs[7 * NBUF + 1 + 2 * NBUF:7 * NBUF + 1 + 3 * NBUF]

        cid = lax.axis_index("c")
        sid = lax.axis_index("s")
        wid = sid * NC + cid
        ebase = wid * EPW

        zero16 = jnp.zeros((16,), jnp.float32)

        for b in range(NBUF):
            for r in range(CHUNK):
                for c9 in range(ACC_W // 16):
                    wvs[b][r, pl.ds(c9 * 16, 16)] = zero16

        for t in range(ROWS_PER_TILE // CHUNK):
            pltpu.sync_copy(
                wvs[0], acc_sh.at[pl.ds(sid * ROWS_PER_TILE + t * CHUNK, CHUNK)])
        plsc.subcore_barrier()

        lane = lax.iota(jnp.int32, 16)
        inv_sqrt = jnp.float32(1.0 / math.sqrt(DH))

        def idx_issue(ci, b):
            base = ebase + ci * CHUNK
            pltpu.async_copy(seg_hbm.at[pl.ds(base, CHUNK)], segs[b], semi[b])
            pltpu.async_copy(oth_hbm.at[pl.ds(base, CHUNK)], oths[b], semi[b])
            pltpu.async_copy(ea_hbm.at[pl.ds(base, CHUNK)], eabs[b], semi[b])

        def idx_wait(b):
            pltpu.make_async_copy(
                seg_hbm.at[pl.ds(0, CHUNK)], segs[b], semi[b]).wait()
            pltpu.make_async_copy(
                oth_hbm.at[pl.ds(0, CHUNK)], oths[b], semi[b]).wait()
            pltpu.make_async_copy(
                ea_hbm.at[pl.ds(0, CHUNK)], eabs[b], semi[b]).wait()

        def gather_issue(b):
            pltpu.async_copy(qg_hbm.at[segs[b]], qgs[b], semg[b])
            pltpu.async_copy(kv_hbm.at[oths[b]], kvs[b], semg[b])

        def gather_wait(b):
            pltpu.make_async_copy(qg_hbm.at[segs[b]], qgs[b], semg[b]).wait()
            pltpu.make_async_copy(kv_hbm.at[oths[b]], kvs[b], semg[b]).wait()

        def compute(b):
            qgv, kvv, eav, wvv = qgs[b], kvs[b], eabs[b], wvs[b]

            def qk_step(s2, accs):
                accs = list(accs)
                for u in range(4):
                    s = s2 * 4 + u
                    rot = jnp.bitwise_and(lane + s, DH - 1)
                    for h in range(HEADS):
                        col = rot + (h * DH)
                        qv = plsc.load_gather(qgv, [lane, col])
                        kx = plsc.load_gather(kvv, [lane, col])
                        accs[h] = accs[h] + qv * kx
                return tuple(accs)

            z = jnp.zeros((16,), jnp.float32)
            accs = lax.fori_loop(0, DH // 4, qk_step, (z, z, z, z))

            def g_step(s2, accs):
                accs = list(accs)
                for u in range(4):
                    s = s2 * 4 + u
                    c16 = jnp.bitwise_and(lane + s, EDGE_DIM - 1)
                    eav_s = plsc.load_gather(eav, [lane, c16])
                    for h in range(HEADS):
                        gv = plsc.load_gather(
                            qgv, [lane, c16 + (DIM + h * EDGE_DIM)])
                        accs[h] = accs[h] + gv * eav_s
                return tuple(accs)

            accs = lax.fori_loop(0, EDGE_DIM // 4, g_step, accs)
            ws = tuple(jnp.exp(a * inv_sqrt) for a in accs)
            for h in range(HEADS):
                plsc.store_scatter(
                    wvv, [lane, jnp.full((16,), DIM + h, jnp.int32)], ws[h])

            def wv_step(s2, wsc):
                for u in range(4):
                    s = s2 * 4 + u
                    rot = jnp.bitwise_and(lane + s, DH - 1)
                    for h in range(HEADS):
                        col = rot + (h * DH)
                        vv = plsc.load_gather(kvv, [lane, col + DIM])
                        plsc.store_scatter(wvv, [lane, col], vv * wsc[h])
                return wsc

            lax.fori_loop(0, DH // 4, wv_step, ws)

        def scatter_issue(b):
            sidx[b][...] = segs[b][...]
            pltpu.async_copy(wvs[b], acc_sh.at[sidx[b]], sems[b], add=True)

        def scatter_wait(b):
            pltpu.make_async_copy(wvs[b], acc_sh.at[sidx[b]], sems[b]).wait()

        idx_issue(0, 0)
        idx_wait(0)
        gather_issue(0)
        idx_issue(1, 1)

        def pipe_body(o, carry):
            for b in range(NBUF):
                ci = o * NBUF + b
                nb = 1 - b
                gather_wait(b)
                idx_wait(nb)
                gather_issue(nb)

                @pl.when(ci >= 2)
                def _():
                    scatter_wait(b)
                compute(b)
                scatter_issue(b)

                @pl.when(ci + 2 <= N_CHUNKS - 1)
                def _():
                    idx_issue(ci + 2, b)
            return carry
        lax.fori_loop(0, (N_CHUNKS - 1) // NBUF, pipe_body, 0)

        gather_wait(0)
        scatter_wait(0)
        compute(0)
        scatter_issue(0)
        scatter_wait(1)
        scatter_wait(0)

        plsc.subcore_barrier()
        for t in range(ROWS_PER_TILE // 160):
            r0 = sid * ROWS_PER_TILE + t * 160
            pltpu.sync_copy(
                acc_sh.at[pl.ds(r0, 160)],
                out_hbm.at[pl.ds(cid * N_PAD + r0, 160)])

    return conv(qg, kv, ea, seg, oth)


ROWS_BLK = 1000
N_BLK = N_NODES // ROWS_BLK
PREP_W = 2 * (QG_W + KV_W)


def _prep(h, W, b):
    def body(h_ref, w_ref, b_ref, qgf_ref, kvf_ref, qgb_ref, kvb_ref):
        res = jnp.dot(h_ref[...], w_ref[...],
                      preferred_element_type=jnp.float32) + b_ref[...]
        o = 0
        qgf_ref[...] = res[:, o:o + QG_W]; o += QG_W
        kvf_ref[...] = res[:, o:o + KV_W]; o += KV_W
        qgb_ref[...] = res[:, o:o + QG_W]; o += QG_W
        kvb_ref[...] = res[:, o:o + KV_W]

    qg_spec = pl.BlockSpec((ROWS_BLK, QG_W), lambda i: (i, 0))
    kv_spec = pl.BlockSpec((ROWS_BLK, KV_W), lambda i: (i, 0))
    qg_shape = jax.ShapeDtypeStruct((N_NODES, QG_W), jnp.float32)
    kv_shape = jax.ShapeDtypeStruct((N_NODES, KV_W), jnp.float32)
    return pl.pallas_call(
        body,
        grid=(N_BLK,),
        in_specs=[
            pl.BlockSpec((ROWS_BLK, DIM), lambda i: (i, 0)),
            pl.BlockSpec((DIM, PREP_W), lambda i: (0, 0)),
            pl.BlockSpec((PREP_W,), lambda i: (0,)),
        ],
        out_specs=[qg_spec, kv_spec, qg_spec, kv_spec],
        out_shape=[qg_shape, kv_shape, qg_shape, kv_shape],
    )(h, W, b)


def _merge(h, accf, accb, mW, ln_g, ln_b, sel, e144):
    def body(h_ref, af0, af1, ab0, ab1, mw_ref, g_ref, b_ref, sel_ref,
             e_ref, o_ref):
        af = af0[0] + af1[0]
        ab = ab0[0] + ab1[0]
        s = sel_ref[...]
        e = e_ref[...]
        fw = jnp.dot(af, s, preferred_element_type=jnp.float32)
        fden = jnp.dot(af, e, preferred_element_type=jnp.float32)
        bw = jnp.dot(ab, s, preferred_element_type=jnp.float32)
        bden = jnp.dot(ab, e, preferred_element_type=jnp.float32)
        fw = fw / (fden + 1e-16)
        bw = bw / (bden + 1e-16)
        mw = mw_ref[...]
        merged = (jnp.dot(h_ref[...], mw[:DIM],
                          preferred_element_type=jnp.float32)
                  + jnp.dot(fw, mw[DIM:2 * DIM],
                            preferred_element_type=jnp.float32)
                  + jnp.dot(bw, mw[2 * DIM:],
                            preferred_element_type=jnp.float32))
        gel = 0.5 * merged * (1.0 + lax.erf(merged * jnp.float32(1.0 / math.sqrt(2.0))))
        mu = jnp.mean(gel, axis=-1, keepdims=True)
        var = jnp.mean((gel - mu) ** 2, axis=-1, keepdims=True)
        o_ref[...] = (gel - mu) / jnp.sqrt(var + 1e-5) * g_ref[...] + b_ref[...]

    acc_spec0 = pl.BlockSpec((1, ROWS_BLK, ACC_W), lambda i: (0, i, 0))
    acc_spec1 = pl.BlockSpec((1, ROWS_BLK, ACC_W), lambda i: (1, i, 0))
    return pl.pallas_call(
        body,
        grid=(N_BLK,),
        in_specs=[
            pl.BlockSpec((ROWS_BLK, DIM), lambda i: (i, 0)),
            acc_spec0, acc_spec1, acc_spec0, acc_spec1,
            pl.BlockSpec((3 * DIM, DIM), lambda i: (0, 0)),
            pl.BlockSpec((DIM,), lambda i: (0,)),
            pl.BlockSpec((DIM,), lambda i: (0,)),
            pl.BlockSpec((ACC_W, DIM), lambda i: (0, 0)),
            pl.BlockSpec((ACC_W, DIM), lambda i: (0, 0)),
        ],
        out_specs=pl.BlockSpec((ROWS_BLK, DIM), lambda i: (i, 0)),
        out_shape=jax.ShapeDtypeStruct((N_NODES, DIM), jnp.float32),
    )(h, accf, accf, accb, accb, mW, ln_g, ln_b, sel, e144)


def _build_kwe(kW):
    kwe_part = kW[DIM:]
    out = jnp.zeros((DIM, HEADS * EDGE_DIM), jnp.float32)
    for hh in range(HEADS):
        blk = kwe_part[:, hh * DH:(hh + 1) * DH].T
        out = out.at[hh * DH:(hh + 1) * DH,
                     hh * EDGE_DIM:(hh + 1) * EDGE_DIM].set(blk)
    return out


def _pack_dir(p):
    kwe = _build_kwe(p["kW"])
    wqg = jnp.concatenate([p["qW"], p["qW"] @ kwe], axis=1)
    bqg = jnp.concatenate([p["qb"], p["qb"] @ kwe], axis=0)
    wkv = jnp.concatenate([p["kW"][:DIM], p["vW"]], axis=1)
    bkv = jnp.concatenate([p["kb"], p["vb"]], axis=0)
    return jnp.concatenate([wqg, wkv], axis=1), jnp.concatenate([bqg, bkv])


def kernel(x, edge_index, edge_attr, params):
    src = edge_index[0]
    dst = edge_index[1]

    sel = jnp.concatenate(
        [jnp.eye(DIM, dtype=jnp.float32),
         jnp.zeros((ACC_W - DIM, DIM), jnp.float32)], axis=0)
    rep = jnp.kron(jnp.eye(HEADS, dtype=jnp.float32),
                   jnp.ones((1, DH), jnp.float32))
    e144 = jnp.concatenate(
        [jnp.zeros((DIM, DIM), jnp.float32), rep,
         jnp.zeros((ACC_W - DIM - HEADS, DIM), jnp.float32)], axis=0)

    h = x
    for p in params["layers"]:
        wf, bf = _pack_dir(p["fw"])
        wb, bb = _pack_dir(p["bw"])
        W = jnp.concatenate([wf, wb], axis=1)
        b = jnp.concatenate([bf, bb])
        qgf, kvf, qgb, kvb = _prep(h, W, b)
        accf = _sc_conv(qgf, kvf, edge_attr, dst, src).reshape(NC, N_PAD, ACC_W)
        accb = _sc_conv(qgb, kvb, edge_attr, src, dst).reshape(NC, N_PAD, ACC_W)
        h = _merge(h, accf, accb, p["mW"], p["ln_g"], p["ln_b"], sel, e144)
    return h


# speedup vs baseline: 1.1298x; 1.1045x over previous
import functools
import math

import jax
import jax.numpy as jnp
from jax import lax
from jax.experimental import pallas as pl
from jax.experimental.pallas import tpu as pltpu
from jax.experimental.pallas import tpu_sc as plsc

N_NODES = 10000
N_EDGES = 320000
DIM = 128
EDGE_DIM = 16
HEADS = 4
DH = DIM // HEADS

QG_W = DIM + HEADS * EDGE_DIM
KV_W = 2 * DIM
ACC_W = 144
QG_P = QG_W // 2
KV_P = KV_W // 2

NC = 2
NS = 16
NW = NC * NS
EPW = N_EDGES // NW
CHUNK = 16
N_CHUNKS = EPW // CHUNK

N_PAD = 10240
ROWS_PER_TILE = N_PAD // NS
ZROWS = 80


NBUF = 2


def _sc_conv(qg, kv, ea, seg, oth):
    mesh = plsc.VectorSubcoreMesh(core_axis_name="c", subcore_axis_name="s")

    scratch = (
        [pltpu.VMEM((CHUNK,), jnp.int32) for _ in range(3 * NBUF)]
        + [pltpu.VMEM((CHUNK, EDGE_DIM), jnp.float32) for _ in range(NBUF)]
        + [pltpu.VMEM((CHUNK, QG_P), jnp.float32) for _ in range(NBUF)]
        + [pltpu.VMEM((CHUNK, KV_P), jnp.float32) for _ in range(NBUF)]
        + [pltpu.VMEM((CHUNK, ACC_W), jnp.float32) for _ in range(NBUF)]
        + [pltpu.VMEM_SHARED((N_PAD, ACC_W), jnp.float32)]
        + [pltpu.SemaphoreType.DMA for _ in range(3 * NBUF)]
    )

    @functools.partial(
        pl.kernel,
        out_type=jax.ShapeDtypeStruct((NC * N_PAD, ACC_W), jnp.float32),
        mesh=mesh,
        compiler_params=pltpu.CompilerParams(
            needs_layout_passes=False, use_tc_tiling_on_sc=False),
        scratch_types=scratch,
    )
    def conv(qg_hbm, kv_hbm, ea_hbm, seg_hbm, oth_hbm, out_hbm, *bufs):
        segs = bufs[0:NBUF]
        oths = bufs[NBUF:2 * NBUF]
        sidx = bufs[2 * NBUF:3 * NBUF]
        eabs = bufs[3 * NBUF:4 * NBUF]
        qgs = bufs[4 * NBUF:5 * NBUF]
        kvs = bufs[5 * NBUF:6 * NBUF]
        wvs = bufs[6 * NBUF:7 * NBUF]
        acc_sh = bufs[7 * NBUF]
        semi = bufs[7 * NBUF + 1:7 * NBUF + 1 + NBUF]
        semg = bufs[7 * NBUF + 1 + NBUF:7 * NBUF + 1 + 2 * NBUF]
        sems = bufs[7 * NBUF + 1 + 2 * NBUF:7 * NBUF + 1 + 3 * NBUF]

        cid = lax.axis_index("c")
        sid = lax.axis_index("s")
        wid = sid * NC + cid
        ebase = wid * EPW

        zero16 = jnp.zeros((16,), jnp.float32)

        for b in range(NBUF):
            for r in range(CHUNK):
                for c9 in range(ACC_W // 16):
                    wvs[b][r, pl.ds(c9 * 16, 16)] = zero16

        for t in range(ROWS_PER_TILE // CHUNK):
            pltpu.sync_copy(
                wvs[0], acc_sh.at[pl.ds(sid * ROWS_PER_TILE + t * CHUNK, CHUNK)])
        plsc.subcore_barrier()

        lane = lax.iota(jnp.int32, 16)
        inv_sqrt = jnp.float32(1.0 / math.sqrt(DH))

        def idx_issue(ci, b):
            base = ebase + ci * CHUNK
            pltpu.async_copy(seg_hbm.at[pl.ds(base, CHUNK)], segs[b], semi[b])
            pltpu.async_copy(oth_hbm.at[pl.ds(base, CHUNK)], oths[b], semi[b])
            pltpu.async_copy(ea_hbm.at[pl.ds(base, CHUNK)], eabs[b], semi[b])

        def idx_wait(b):
            pltpu.make_async_copy(
                seg_hbm.at[pl.ds(0, CHUNK)], segs[b], semi[b]).wait()
            pltpu.make_async_copy(
                oth_hbm.at[pl.ds(0, CHUNK)], oths[b], semi[b]).wait()
            pltpu.make_async_copy(
                ea_hbm.at[pl.ds(0, CHUNK)], eabs[b], semi[b]).wait()

        def gather_issue(b):
            pltpu.async_copy(qg_hbm.at[segs[b]], qgs[b], semg[b])
            pltpu.async_copy(kv_hbm.at[oths[b]], kvs[b], semg[b])

        def gather_wait(b):
            pltpu.make_async_copy(qg_hbm.at[segs[b]], qgs[b], semg[b]).wait()
            pltpu.make_async_copy(kv_hbm.at[oths[b]], kvs[b], semg[b]).wait()

        def compute(b):
            qgv, kvv, eav, wvv = qgs[b], kvs[b], eabs[b], wvs[b]

            def unpk(p):
                return plsc.unpack(plsc.bitcast(p, jnp.bfloat16),
                                   format=plsc.PackFormat.INTERLEAVED,
                                   preferred_element_type=jnp.float32)

            def qk_step(s4, accs):
                accs = list(accs)
                for u in range(4):
                    s = s4 * 4 + u
                    rotp = jnp.bitwise_and(lane + s, DH // 2 - 1)
                    for h in range(HEADS):
                        colp = rotp + (h * (DH // 2))
                        qe, qo = unpk(plsc.load_gather(qgv, [lane, colp]))
                        ke, ko = unpk(plsc.load_gather(kvv, [lane, colp]))
                        accs[h] = accs[h] + qe * ke + qo * ko
                return tuple(accs)

            z = jnp.zeros((16,), jnp.float32)
            accs = lax.fori_loop(0, DH // 8, qk_step, (z, z, z, z))

            def g_step(s4, accs):
                accs = list(accs)
                for u in range(4):
                    s = s4 * 4 + u
                    c8 = jnp.bitwise_and(lane + s, EDGE_DIM // 2 - 1)
                    eae = plsc.load_gather(eav, [lane, 2 * c8])
                    eao = plsc.load_gather(eav, [lane, 2 * c8 + 1])
                    for h in range(HEADS):
                        gp = plsc.load_gather(
                            qgv, [lane, c8 + (DIM // 2 + h * (EDGE_DIM // 2))])
                        ge, go = unpk(gp)
                        accs[h] = accs[h] + ge * eae + go * eao
                return tuple(accs)

            accs = lax.fori_loop(0, EDGE_DIM // 8, g_step, accs)
            ws = tuple(jnp.exp(a * inv_sqrt) for a in accs)
            for h in range(HEADS):
                plsc.store_scatter(
                    wvv, [lane, jnp.full((16,), DIM + h, jnp.int32)], ws[h])

            def wv_step(s4, wsc):
                for u in range(4):
                    s = s4 * 4 + u
                    rotp = jnp.bitwise_and(lane + s, DH // 2 - 1)
                    for h in range(HEADS):
                        colp = rotp + (DIM // 2 + h * (DH // 2))
                        ve, vo = unpk(plsc.load_gather(kvv, [lane, colp]))
                        dcol = 2 * rotp + (h * DH)
                        plsc.store_scatter(wvv, [lane, dcol], ve * wsc[h])
                        plsc.store_scatter(wvv, [lane, dcol + 1], vo * wsc[h])
                return wsc

            lax.fori_loop(0, DH // 8, wv_step, ws)

        def scatter_issue(b):
            sidx[b][...] = segs[b][...]
            pltpu.async_copy(wvs[b], acc_sh.at[sidx[b]], sems[b], add=True)

        def scatter_wait(b):
            pltpu.make_async_copy(wvs[b], acc_sh.at[sidx[b]], sems[b]).wait()

        idx_issue(0, 0)
        idx_wait(0)
        gather_issue(0)
        idx_issue(1, 1)

        def pipe_body(o, carry):
            for b in range(NBUF):
                ci = o * NBUF + b
                nb = 1 - b
                gather_wait(b)
                idx_wait(nb)
                gather_issue(nb)

                @pl.when(ci >= 2)
                def _():
                    scatter_wait(b)
                compute(b)
                scatter_issue(b)

                @pl.when(ci + 2 <= N_CHUNKS - 1)
                def _():
                    idx_issue(ci + 2, b)
            return carry
        lax.fori_loop(0, (N_CHUNKS - 1) // NBUF, pipe_body, 0)

        gather_wait(0)
        scatter_wait(0)
        compute(0)
        scatter_issue(0)
        scatter_wait(1)
        scatter_wait(0)

        plsc.subcore_barrier()
        for t in range(ROWS_PER_TILE // 160):
            r0 = sid * ROWS_PER_TILE + t * 160
            pltpu.sync_copy(
                acc_sh.at[pl.ds(r0, 160)],
                out_hbm.at[pl.ds(cid * N_PAD + r0, 160)])

    return conv(qg, kv, ea, seg, oth)


ROWS_BLK = 1000
N_BLK = N_NODES // ROWS_BLK
PREP_W = 2 * (QG_W + KV_W)


def _prep(h, W, b):
    def body(h_ref, w_ref, b_ref, qgf_ref, kvf_ref, qgb_ref, kvb_ref):
        res = jnp.dot(h_ref[...], w_ref[...],
                      preferred_element_type=jnp.float32) + b_ref[...]
        o = 0
        qgf_ref[...] = res[:, o:o + QG_W].astype(jnp.bfloat16); o += QG_W
        kvf_ref[...] = res[:, o:o + KV_W].astype(jnp.bfloat16); o += KV_W
        qgb_ref[...] = res[:, o:o + QG_W].astype(jnp.bfloat16); o += QG_W
        kvb_ref[...] = res[:, o:o + KV_W].astype(jnp.bfloat16)

    qg_spec = pl.BlockSpec((ROWS_BLK, QG_W), lambda i: (i, 0))
    kv_spec = pl.BlockSpec((ROWS_BLK, KV_W), lambda i: (i, 0))
    qg_shape = jax.ShapeDtypeStruct((N_NODES, QG_W), jnp.bfloat16)
    kv_shape = jax.ShapeDtypeStruct((N_NODES, KV_W), jnp.bfloat16)
    return pl.pallas_call(
        body,
        grid=(N_BLK,),
        in_specs=[
            pl.BlockSpec((ROWS_BLK, DIM), lambda i: (i, 0)),
            pl.BlockSpec((DIM, PREP_W), lambda i: (0, 0)),
            pl.BlockSpec((PREP_W,), lambda i: (0,)),
        ],
        out_specs=[qg_spec, kv_spec, qg_spec, kv_spec],
        out_shape=[qg_shape, kv_shape, qg_shape, kv_shape],
    )(h, W, b)


def _merge(h, accf, accb, mW, ln_g, ln_b, sel, e144):
    def body(h_ref, af0, af1, ab0, ab1, mw_ref, g_ref, b_ref, sel_ref,
             e_ref, o_ref):
        af = af0[0] + af1[0]
        ab = ab0[0] + ab1[0]
        s = sel_ref[...]
        e = e_ref[...]
        fw = jnp.dot(af, s, preferred_element_type=jnp.float32)
        fden = jnp.dot(af, e, preferred_element_type=jnp.float32)
        bw = jnp.dot(ab, s, preferred_element_type=jnp.float32)
        bden = jnp.dot(ab, e, preferred_element_type=jnp.float32)
        fw = fw / (fden + 1e-16)
        bw = bw / (bden + 1e-16)
        mw = mw_ref[...]
        merged = (jnp.dot(h_ref[...], mw[:DIM],
                          preferred_element_type=jnp.float32)
                  + jnp.dot(fw, mw[DIM:2 * DIM],
                            preferred_element_type=jnp.float32)
                  + jnp.dot(bw, mw[2 * DIM:],
                            preferred_element_type=jnp.float32))
        gel = 0.5 * merged * (1.0 + lax.erf(merged * jnp.float32(1.0 / math.sqrt(2.0))))
        mu = jnp.mean(gel, axis=-1, keepdims=True)
        var = jnp.mean((gel - mu) ** 2, axis=-1, keepdims=True)
        o_ref[...] = (gel - mu) / jnp.sqrt(var + 1e-5) * g_ref[...] + b_ref[...]

    acc_spec0 = pl.BlockSpec((1, ROWS_BLK, ACC_W), lambda i: (0, i, 0))
    acc_spec1 = pl.BlockSpec((1, ROWS_BLK, ACC_W), lambda i: (1, i, 0))
    return pl.pallas_call(
        body,
        grid=(N_BLK,),
        in_specs=[
            pl.BlockSpec((ROWS_BLK, DIM), lambda i: (i, 0)),
            acc_spec0, acc_spec1, acc_spec0, acc_spec1,
            pl.BlockSpec((3 * DIM, DIM), lambda i: (0, 0)),
            pl.BlockSpec((DIM,), lambda i: (0,)),
            pl.BlockSpec((DIM,), lambda i: (0,)),
            pl.BlockSpec((ACC_W, DIM), lambda i: (0, 0)),
            pl.BlockSpec((ACC_W, DIM), lambda i: (0, 0)),
        ],
        out_specs=pl.BlockSpec((ROWS_BLK, DIM), lambda i: (i, 0)),
        out_shape=jax.ShapeDtypeStruct((N_NODES, DIM), jnp.float32),
    )(h, accf, accf, accb, accb, mW, ln_g, ln_b, sel, e144)


def _build_kwe(kW):
    kwe_part = kW[DIM:]
    out = jnp.zeros((DIM, HEADS * EDGE_DIM), jnp.float32)
    for hh in range(HEADS):
        blk = kwe_part[:, hh * DH:(hh + 1) * DH].T
        out = out.at[hh * DH:(hh + 1) * DH,
                     hh * EDGE_DIM:(hh + 1) * EDGE_DIM].set(blk)
    return out


def _pack_dir(p):
    kwe = _build_kwe(p["kW"])
    wqg = jnp.concatenate([p["qW"], p["qW"] @ kwe], axis=1)
    bqg = jnp.concatenate([p["qb"], p["qb"] @ kwe], axis=0)
    wkv = jnp.concatenate([p["kW"][:DIM], p["vW"]], axis=1)
    bkv = jnp.concatenate([p["kb"], p["vb"]], axis=0)
    return jnp.concatenate([wqg, wkv], axis=1), jnp.concatenate([bqg, bkv])


def kernel(x, edge_index, edge_attr, params):
    src = edge_index[0]
    dst = edge_index[1]

    sel = jnp.concatenate(
        [jnp.eye(DIM, dtype=jnp.float32),
         jnp.zeros((ACC_W - DIM, DIM), jnp.float32)], axis=0)
    rep = jnp.kron(jnp.eye(HEADS, dtype=jnp.float32),
                   jnp.ones((1, DH), jnp.float32))
    e144 = jnp.concatenate(
        [jnp.zeros((DIM, DIM), jnp.float32), rep,
         jnp.zeros((ACC_W - DIM - HEADS, DIM), jnp.float32)], axis=0)

    h = x
    for p in params["layers"]:
        wf, bf = _pack_dir(p["fw"])
        wb, bb = _pack_dir(p["bw"])
        W = jnp.concatenate([wf, wb], axis=1)
        b = jnp.concatenate([bf, bb])
        qgf, kvf, qgb, kvb = _prep(h, W, b)
        qgf = lax.bitcast_convert_type(qgf.reshape(N_NODES, QG_P, 2), jnp.float32)
        kvf = lax.bitcast_convert_type(kvf.reshape(N_NODES, KV_P, 2), jnp.float32)
        qgb = lax.bitcast_convert_type(qgb.reshape(N_NODES, QG_P, 2), jnp.float32)
        kvb = lax.bitcast_convert_type(kvb.reshape(N_NODES, KV_P, 2), jnp.float32)
        accf = _sc_conv(qgf, kvf, edge_attr, dst, src).reshape(NC, N_PAD, ACC_W)
        accb = _sc_conv(qgb, kvb, edge_attr, src, dst).reshape(NC, N_PAD, ACC_W)
        h = _merge(h, accf, accb, p["mW"], p["ln_g"], p["ln_b"], sel, e144)
    return h
